# Initial kernel scaffold; baseline (speedup 1.0000x reference)
import jax
import jax.numpy as jnp
from jax.experimental import pallas as pl


def kernel(x, params, idx_k8):
    raise NotImplementedError("write your pallas kernel here")


# speedup vs baseline: 2.2930x; 2.2930x over previous
import functools

import jax
import jax.numpy as jnp
import numpy as np
from jax import lax
from jax.experimental import pallas as pl
from jax.experimental.pallas import tpu as pltpu
from jax.experimental.pallas import tpu_sc as plsc

N = 50000
D = 128
H = 4
DH = 32
K = 8
FF = 512
OUT = 128

NW = 32
NP = 50176
PW = NP // NW
IPW = PW * K
GCH = 128
NCHUNK = IPW // GCH

BLK = 512
ABLK = 256


def _lnorm(h, s, b):
    m = jnp.mean(h, axis=-1, keepdims=True)
    v = jnp.mean((h - m) ** 2, axis=-1, keepdims=True)
    return (h - m) * lax.rsqrt(v + 1e-5) * s + b


def _row_spec(blk, width):
    return pl.BlockSpec((blk, width), lambda i: (i, 0))


def _full_spec(shape):
    return pl.BlockSpec(shape, lambda i: tuple(0 for _ in shape))


def _enc_body(x_ref, w1, b1, w2, b2, ls, lb, o_ref):
    h = jnp.maximum(x_ref[...] @ w1[...] + b1[...], 0.0)
    h = h @ w2[...] + b2[...]
    o_ref[...] = _lnorm(h, ls[...], lb[...])


def _enc(x, e):
    grid = (NP // BLK,)
    return pl.pallas_call(
        _enc_body,
        grid=grid,
        in_specs=[
            _row_spec(BLK, D),
            _full_spec((D, D)), _full_spec((1, D)),
            _full_spec((D, D)), _full_spec((1, D)),
            _full_spec((1, D)), _full_spec((1, D)),
        ],
        out_specs=_row_spec(BLK, D),
        out_shape=jax.ShapeDtypeStruct((NP, D), jnp.float32),
    )(x, e['W1'], e['b1'].reshape(1, D), e['W2'], e['b2'].reshape(1, D),
      e['ln_s'].reshape(1, D), e['ln_b'].reshape(1, D))


def _qkv_body(x_ref, ls, lb, wq, bq, wk, bk, wv, bv, q_ref, k_ref, v_ref):
    h = _lnorm(x_ref[...], ls[...], lb[...])
    q_ref[...] = h @ wq[...] + bq[...]
    k_ref[...] = h @ wk[...] + bk[...]
    v_ref[...] = h @ wv[...] + bv[...]


def _qkv(x, p):
    grid = (NP // BLK,)
    shp = jax.ShapeDtypeStruct((NP, D), jnp.float32)
    return pl.pallas_call(
        _qkv_body,
        grid=grid,
        in_specs=[
            _row_spec(BLK, D),
            _full_spec((1, D)), _full_spec((1, D)),
            _full_spec((D, D)), _full_spec((1, D)),
            _full_spec((D, D)), _full_spec((1, D)),
            _full_spec((D, D)), _full_spec((1, D)),
        ],
        out_specs=[_row_spec(BLK, D)] * 3,
        out_shape=[shp, shp, shp],
    )(x, p['ln1_s'].reshape(1, D), p['ln1_b'].reshape(1, D),
      p['Wq'], p['bq'].reshape(1, D), p['Wk'], p['bk'].reshape(1, D),
      p['Wv'], p['bv'].reshape(1, D))


def _attn_body(q_ref, ks_ref, vs_ref, kg_ref, vg_ref, o_ref):
    B = q_ref.shape[0]
    q = q_ref[...]
    ks = ks_ref[...]
    vs = vs_ref[...]
    kg = kg_ref[...].reshape(B, K, D)
    vg = vg_ref[...].reshape(B, K, D)
    r = lax.broadcasted_iota(jnp.int32, (D, H), 0) // DH
    c = lax.broadcasted_iota(jnp.int32, (D, H), 1)
    S = (r == c).astype(jnp.float32)
    r2 = lax.broadcasted_iota(jnp.int32, (H, D), 0)
    c2 = lax.broadcasted_iota(jnp.int32, (H, D), 1) // DH
    ST = (r2 == c2).astype(jnp.float32)
    scale = np.float32(1.0 / np.sqrt(DH))
    ls = ((q * ks) @ S) * scale
    prod = (q[:, None, :] * kg).reshape(B * K, D)
    ln_ = (prod @ S).reshape(B, K, H) * scale
    m = jnp.maximum(jnp.max(ln_, axis=1), ls)
    es = jnp.exp(ls - m)
    en = jnp.exp(ln_ - m[:, None, :])
    ssum = es + jnp.sum(en, axis=1)
    a_self = es / ssum
    a_n = en / ssum[:, None, :]
    af = (a_n.reshape(B * K, H) @ ST).reshape(B, K, D)
    o_ref[...] = (a_self @ ST) * vs + jnp.sum(af * vg, axis=1)


def _attn(q, ks, vs, kg, vg):
    grid = (NP // ABLK,)
    return pl.pallas_call(
        _attn_body,
        grid=grid,
        in_specs=[
            _row_spec(ABLK, D), _row_spec(ABLK, D), _row_spec(ABLK, D),
            _row_spec(ABLK * K, D), _row_spec(ABLK * K, D),
        ],
        out_specs=_row_spec(ABLK, D),
        out_shape=jax.ShapeDtypeStruct((NP, D), jnp.float32),
    )(q, ks, vs, kg, vg)


def _post_body(x_ref, o_in, wo, bo, l2s, l2b, w1, b1, w2, b2, y_ref):
    x2 = x_ref[...] + o_in[...] @ wo[...] + bo[...]
    h2 = _lnorm(x2, l2s[...], l2b[...])
    y_ref[...] = x2 + jnp.maximum(h2 @ w1[...] + b1[...], 0.0) @ w2[...] + b2[...]


def _post(x, o, p):
    grid = (NP // BLK,)
    return pl.pallas_call(
        _post_body,
        grid=grid,
        in_specs=[
            _row_spec(BLK, D), _row_spec(BLK, D),
            _full_spec((D, D)), _full_spec((1, D)),
            _full_spec((1, D)), _full_spec((1, D)),
            _full_spec((D, FF)), _full_spec((1, FF)),
            _full_spec((FF, D)), _full_spec((1, D)),
        ],
        out_specs=_row_spec(BLK, D),
        out_shape=jax.ShapeDtypeStruct((NP, D), jnp.float32),
    )(x, o, p['Wo'], p['bo'].reshape(1, D),
      p['ln2_s'].reshape(1, D), p['ln2_b'].reshape(1, D),
      p['W1'], p['b1'].reshape(1, FF), p['W2'], p['b2'].reshape(1, D))


def _dec_body(x_ref, w1, b1, w2, b2, y_ref):
    h = jnp.maximum(x_ref[...] @ w1[...] + b1[...], 0.0)
    y_ref[...] = h @ w2[...] + b2[...]


def _dec(x, d):
    grid = (NP // BLK,)
    return pl.pallas_call(
        _dec_body,
        grid=grid,
        in_specs=[
            _row_spec(BLK, D),
            _full_spec((D, D)), _full_spec((1, D)),
            _full_spec((D, OUT)), _full_spec((1, OUT)),
        ],
        out_specs=_row_spec(BLK, OUT),
        out_shape=jax.ShapeDtypeStruct((NP, OUT), jnp.float32),
    )(x, d['W1'], d['b1'].reshape(1, D), d['W2'], d['b2'].reshape(1, OUT))


def _sc_gather_body(k_hbm, v_hbm, idx_hbm, kg_hbm, vg_hbm,
                    idx_v, kr, vr, s1, s2):
    wid = lax.axis_index("s") * 2 + lax.axis_index("c")
    base = wid * IPW

    def body(i, carry):
        off = base + i * GCH
        pltpu.sync_copy(idx_hbm.at[pl.ds(off, GCH)], idx_v)
        ck = pltpu.async_copy(k_hbm.at[idx_v], kr, s1)
        cv = pltpu.async_copy(v_hbm.at[idx_v], vr, s2)
        ck.wait()
        cv.wait()
        pltpu.sync_copy(kr, kg_hbm.at[pl.ds(off, GCH)])
        pltpu.sync_copy(vr, vg_hbm.at[pl.ds(off, GCH)])
        return carry

    lax.fori_loop(0, NCHUNK, body, 0)


def _sc_gather(k_all, v_all, idx_flat):
    mesh = plsc.VectorSubcoreMesh(core_axis_name="c", subcore_axis_name="s",
                                  num_cores=2, num_subcores=16)
    shp = jax.ShapeDtypeStruct((NP * K, D), jnp.float32)
    fn = pl.kernel(
        _sc_gather_body,
        out_type=(shp, shp),
        mesh=mesh,
        scratch_types=[
            pltpu.VMEM((GCH,), jnp.int32),
            pltpu.VMEM((GCH, D), jnp.float32),
            pltpu.VMEM((GCH, D), jnp.float32),
            pltpu.SemaphoreType.DMA,
            pltpu.SemaphoreType.DMA,
        ],
    )
    return fn(k_all, v_all, idx_flat)


def kernel(x, params, idx_k8):
    xp = jnp.pad(x, ((0, NP - N), (0, 0)))
    idx_flat = jnp.pad(idx_k8, ((0, NP - N), (0, 0))).reshape(NP * K)
    h = _enc(xp, params['enc'])
    for p in params['blocks']:
        q, k_all, v_all = _qkv(h, p)
        kg, vg = _sc_gather(k_all, v_all, idx_flat)
        o = _attn(q, k_all, v_all, kg, vg)
        h = _post(h, o, p)
    out = _dec(h, params['dec'])
    return out[:N]


# speedup vs baseline: 2.5266x; 1.1019x over previous
import functools

import jax
import jax.numpy as jnp
import numpy as np
from jax import lax
from jax.experimental import pallas as pl
from jax.experimental.pallas import tpu as pltpu
from jax.experimental.pallas import tpu_sc as plsc

N = 50000
D = 128
H = 4
DH = 32
K = 8
FF = 512
OUT = 128

NW = 32
NP = 50176
PW = NP // NW
IPW = PW * K
GCH = 128
NCHUNK = IPW // GCH

BLK = 512
ABLK = 256


def _lnorm(h, s, b):
    m = jnp.mean(h, axis=-1, keepdims=True)
    v = jnp.mean((h - m) ** 2, axis=-1, keepdims=True)
    return (h - m) * lax.rsqrt(v + 1e-5) * s + b


def _row_spec(blk, width):
    return pl.BlockSpec((blk, width), lambda i: (i, 0))


def _full_spec(shape):
    return pl.BlockSpec(shape, lambda i: tuple(0 for _ in shape))


def _enc_body(x_ref, w1, b1, w2, b2, ls, lb, o_ref):
    h = jnp.maximum(x_ref[...] @ w1[...] + b1[...], 0.0)
    h = h @ w2[...] + b2[...]
    o_ref[...] = _lnorm(h, ls[...], lb[...])


def _enc(x, e):
    grid = (NP // BLK,)
    return pl.pallas_call(
        _enc_body,
        grid=grid,
        in_specs=[
            _row_spec(BLK, D),
            _full_spec((D, D)), _full_spec((1, D)),
            _full_spec((D, D)), _full_spec((1, D)),
            _full_spec((1, D)), _full_spec((1, D)),
        ],
        out_specs=_row_spec(BLK, D),
        out_shape=jax.ShapeDtypeStruct((NP, D), jnp.float32),
    )(x, e['W1'], e['b1'].reshape(1, D), e['W2'], e['b2'].reshape(1, D),
      e['ln_s'].reshape(1, D), e['ln_b'].reshape(1, D))


def _qkv_body(x_ref, ls, lb, wq, bq, wk, bk, wv, bv, q_ref, k_ref, v_ref):
    h = _lnorm(x_ref[...], ls[...], lb[...])
    q_ref[...] = h @ wq[...] + bq[...]
    k_ref[...] = h @ wk[...] + bk[...]
    v_ref[...] = h @ wv[...] + bv[...]


def _qkv(x, p):
    grid = (NP // BLK,)
    shp = jax.ShapeDtypeStruct((NP, D), jnp.float32)
    return pl.pallas_call(
        _qkv_body,
        grid=grid,
        in_specs=[
            _row_spec(BLK, D),
            _full_spec((1, D)), _full_spec((1, D)),
            _full_spec((D, D)), _full_spec((1, D)),
            _full_spec((D, D)), _full_spec((1, D)),
            _full_spec((D, D)), _full_spec((1, D)),
        ],
        out_specs=[_row_spec(BLK, D)] * 3,
        out_shape=[shp, shp, shp],
    )(x, p['ln1_s'].reshape(1, D), p['ln1_b'].reshape(1, D),
      p['Wq'], p['bq'].reshape(1, D), p['Wk'], p['bk'].reshape(1, D),
      p['Wv'], p['bv'].reshape(1, D))


def _attn_body(q_ref, ks_ref, vs_ref, kg_ref, vg_ref, o_ref):
    B = q_ref.shape[0]
    q = q_ref[...]
    ks = ks_ref[...]
    vs = vs_ref[...]
    kg = kg_ref[...].reshape(B, K, D)
    vg = vg_ref[...].reshape(B, K, D)
    r = lax.broadcasted_iota(jnp.int32, (D, H), 0) // DH
    c = lax.broadcasted_iota(jnp.int32, (D, H), 1)
    S = (r == c).astype(jnp.float32)
    r2 = lax.broadcasted_iota(jnp.int32, (H, D), 0)
    c2 = lax.broadcasted_iota(jnp.int32, (H, D), 1) // DH
    ST = (r2 == c2).astype(jnp.float32)
    scale = np.float32(1.0 / np.sqrt(DH))
    ls = ((q * ks) @ S) * scale
    prod = (q[:, None, :] * kg).reshape(B * K, D)
    ln_ = (prod @ S).reshape(B, K, H) * scale
    m = jnp.maximum(jnp.max(ln_, axis=1), ls)
    es = jnp.exp(ls - m)
    en = jnp.exp(ln_ - m[:, None, :])
    ssum = es + jnp.sum(en, axis=1)
    a_self = es / ssum
    a_n = en / ssum[:, None, :]
    af = (a_n.reshape(B * K, H) @ ST).reshape(B, K, D)
    o_ref[...] = (a_self @ ST) * vs + jnp.sum(af * vg, axis=1)


def _attn(q, ks, vs, kg, vg):
    grid = (NP // ABLK,)
    return pl.pallas_call(
        _attn_body,
        grid=grid,
        in_specs=[
            _row_spec(ABLK, D), _row_spec(ABLK, D), _row_spec(ABLK, D),
            _row_spec(ABLK * K, D), _row_spec(ABLK * K, D),
        ],
        out_specs=_row_spec(ABLK, D),
        out_shape=jax.ShapeDtypeStruct((NP, D), jnp.float32),
    )(q, ks, vs, kg, vg)


def _post_body(x_ref, o_in, wo, bo, l2s, l2b, w1, b1, w2, b2, y_ref):
    x2 = x_ref[...] + o_in[...] @ wo[...] + bo[...]
    h2 = _lnorm(x2, l2s[...], l2b[...])
    y_ref[...] = x2 + jnp.maximum(h2 @ w1[...] + b1[...], 0.0) @ w2[...] + b2[...]


def _post(x, o, p):
    grid = (NP // BLK,)
    return pl.pallas_call(
        _post_body,
        grid=grid,
        in_specs=[
            _row_spec(BLK, D), _row_spec(BLK, D),
            _full_spec((D, D)), _full_spec((1, D)),
            _full_spec((1, D)), _full_spec((1, D)),
            _full_spec((D, FF)), _full_spec((1, FF)),
            _full_spec((FF, D)), _full_spec((1, D)),
        ],
        out_specs=_row_spec(BLK, D),
        out_shape=jax.ShapeDtypeStruct((NP, D), jnp.float32),
    )(x, o, p['Wo'], p['bo'].reshape(1, D),
      p['ln2_s'].reshape(1, D), p['ln2_b'].reshape(1, D),
      p['W1'], p['b1'].reshape(1, FF), p['W2'], p['b2'].reshape(1, D))


def _dec_body(x_ref, w1, b1, w2, b2, y_ref):
    h = jnp.maximum(x_ref[...] @ w1[...] + b1[...], 0.0)
    y_ref[...] = h @ w2[...] + b2[...]


def _dec(x, d):
    grid = (NP // BLK,)
    return pl.pallas_call(
        _dec_body,
        grid=grid,
        in_specs=[
            _row_spec(BLK, D),
            _full_spec((D, D)), _full_spec((1, D)),
            _full_spec((D, OUT)), _full_spec((1, OUT)),
        ],
        out_specs=_row_spec(BLK, OUT),
        out_shape=jax.ShapeDtypeStruct((NP, OUT), jnp.float32),
    )(x, d['W1'], d['b1'].reshape(1, D), d['W2'], d['b2'].reshape(1, OUT))


NB = 3


def _sc_gather_body(k_hbm, v_hbm, idx_hbm, kg_hbm, vg_hbm,
                    idx_v, kr, vr, *sems):
    sgk, sgv, sok, sov = sems[0:NB], sems[NB:2 * NB], sems[2 * NB:3 * NB], sems[3 * NB:4 * NB]
    wid = lax.axis_index("s") * 2 + lax.axis_index("c")
    base = wid * IPW
    pltpu.sync_copy(idx_hbm.at[pl.ds(base, IPW)], idx_v)

    pend_g = {}
    pend_o = {}
    for i in range(NCHUNK + 1):
        if i < NCHUNK:
            b = i % NB
            if i >= NB:
                ok, ov = pend_o.pop(i - NB)
                ok.wait()
                ov.wait()
            ii = pl.ds(i * GCH, GCH)
            gk = pltpu.async_copy(k_hbm.at[idx_v.at[ii]], kr.at[b], sgk[b])
            gv = pltpu.async_copy(v_hbm.at[idx_v.at[ii]], vr.at[b], sgv[b])
            pend_g[i] = (gk, gv)
        if i >= 1:
            j = i - 1
            b = j % NB
            gk, gv = pend_g.pop(j)
            gk.wait()
            gv.wait()
            off = pl.ds(base + j * GCH, GCH)
            ok = pltpu.async_copy(kr.at[b], kg_hbm.at[off], sok[b])
            ov = pltpu.async_copy(vr.at[b], vg_hbm.at[off], sov[b])
            pend_o[j] = (ok, ov)
    for j in sorted(pend_o):
        ok, ov = pend_o[j]
        ok.wait()
        ov.wait()


def _sc_gather(k_all, v_all, idx_flat):
    mesh = plsc.VectorSubcoreMesh(core_axis_name="c", subcore_axis_name="s",
                                  num_cores=2, num_subcores=16)
    shp = jax.ShapeDtypeStruct((NP * K, D), jnp.float32)
    fn = pl.kernel(
        _sc_gather_body,
        out_type=(shp, shp),
        mesh=mesh,
        scratch_types=[
            pltpu.VMEM((IPW,), jnp.int32),
            pltpu.VMEM((NB, GCH, D), jnp.float32),
            pltpu.VMEM((NB, GCH, D), jnp.float32),
        ] + [pltpu.SemaphoreType.DMA] * (4 * NB),
    )
    return fn(k_all, v_all, idx_flat)


def kernel(x, params, idx_k8):
    xp = jnp.pad(x, ((0, NP - N), (0, 0)))
    idx_flat = jnp.pad(idx_k8, ((0, NP - N), (0, 0))).reshape(NP * K)
    h = _enc(xp, params['enc'])
    for p in params['blocks']:
        q, k_all, v_all = _qkv(h, p)
        kg, vg = _sc_gather(k_all, v_all, idx_flat)
        o = _attn(q, k_all, v_all, kg, vg)
        h = _post(h, o, p)
    out = _dec(h, params['dec'])
    return out[:N]


# speedup vs baseline: 3.0161x; 1.1937x over previous
import functools

import jax
import jax.numpy as jnp
import numpy as np
from jax import lax
from jax.experimental import pallas as pl
from jax.experimental.pallas import tpu as pltpu
from jax.experimental.pallas import tpu_sc as plsc

N = 50000
D = 128
H = 4
DH = 32
K = 8
FF = 512
OUT = 128

NW = 32
NP = 50176
PW = NP // NW
IPW = PW * K
GCH = 128
NCHUNK = IPW // GCH

BLK = 512
ABLK = 256


def _lnorm(h, s, b):
    m = jnp.mean(h, axis=-1, keepdims=True)
    v = jnp.mean((h - m) ** 2, axis=-1, keepdims=True)
    return (h - m) * lax.rsqrt(v + 1e-5) * s + b


def _row_spec(blk, width):
    return pl.BlockSpec((blk, width), lambda i: (i, 0))


def _full_spec(shape):
    return pl.BlockSpec(shape, lambda i: tuple(0 for _ in shape))


def _enc_body(x_ref, w1, b1, w2, b2, ls, lb, o_ref):
    h = jnp.maximum(x_ref[...] @ w1[...] + b1[...], 0.0)
    h = h @ w2[...] + b2[...]
    o_ref[...] = _lnorm(h, ls[...], lb[...])


def _enc(x, e):
    grid = (NP // BLK,)
    return pl.pallas_call(
        _enc_body,
        grid=grid,
        in_specs=[
            _row_spec(BLK, D),
            _full_spec((D, D)), _full_spec((1, D)),
            _full_spec((D, D)), _full_spec((1, D)),
            _full_spec((1, D)), _full_spec((1, D)),
        ],
        out_specs=_row_spec(BLK, D),
        out_shape=jax.ShapeDtypeStruct((NP, D), jnp.float32),
    )(x, e['W1'], e['b1'].reshape(1, D), e['W2'], e['b2'].reshape(1, D),
      e['ln_s'].reshape(1, D), e['ln_b'].reshape(1, D))


def _qkv_body(x_ref, ls, lb, wq, bq, wk, bk, wv, bv,
              q_ref, k_ref, v_ref, kv_ref):
    h = _lnorm(x_ref[...], ls[...], lb[...])
    q_ref[...] = h @ wq[...] + bq[...]
    k = h @ wk[...] + bk[...]
    v = h @ wv[...] + bv[...]
    k_ref[...] = k
    v_ref[...] = v
    kw = lax.bitcast_convert_type(k.astype(jnp.bfloat16), jnp.uint16).astype(jnp.uint32)
    vw = lax.bitcast_convert_type(v.astype(jnp.bfloat16), jnp.uint16).astype(jnp.uint32)
    kv_ref[...] = ((kw << 16) | vw).astype(jnp.int32)


def _qkv(x, p):
    grid = (NP // BLK,)
    shp = jax.ShapeDtypeStruct((NP, D), jnp.float32)
    shi = jax.ShapeDtypeStruct((NP, D), jnp.int32)
    return pl.pallas_call(
        _qkv_body,
        grid=grid,
        in_specs=[
            _row_spec(BLK, D),
            _full_spec((1, D)), _full_spec((1, D)),
            _full_spec((D, D)), _full_spec((1, D)),
            _full_spec((D, D)), _full_spec((1, D)),
            _full_spec((D, D)), _full_spec((1, D)),
        ],
        out_specs=[_row_spec(BLK, D)] * 4,
        out_shape=[shp, shp, shp, shi],
    )(x, p['ln1_s'].reshape(1, D), p['ln1_b'].reshape(1, D),
      p['Wq'], p['bq'].reshape(1, D), p['Wk'], p['bk'].reshape(1, D),
      p['Wv'], p['bv'].reshape(1, D))


def _attn_body(q_ref, ks_ref, vs_ref, kvg_ref, o_ref):
    B = q_ref.shape[0]
    q = q_ref[...]
    ks = ks_ref[...]
    vs = vs_ref[...]
    kvw = lax.bitcast_convert_type(kvg_ref[...], jnp.uint32)
    kg = lax.bitcast_convert_type((kvw >> 16).astype(jnp.uint16), jnp.bfloat16)
    vg = lax.bitcast_convert_type(kvw.astype(jnp.uint16), jnp.bfloat16)
    kg = kg.astype(jnp.float32).reshape(B, K, D)
    vg = vg.astype(jnp.float32).reshape(B, K, D)
    r = lax.broadcasted_iota(jnp.int32, (D, H), 0) // DH
    c = lax.broadcasted_iota(jnp.int32, (D, H), 1)
    S = (r == c).astype(jnp.float32)
    r2 = lax.broadcasted_iota(jnp.int32, (H, D), 0)
    c2 = lax.broadcasted_iota(jnp.int32, (H, D), 1) // DH
    ST = (r2 == c2).astype(jnp.float32)
    scale = np.float32(1.0 / np.sqrt(DH))
    ls = ((q * ks) @ S) * scale
    prod = (q[:, None, :] * kg).reshape(B * K, D)
    ln_ = (prod @ S).reshape(B, K, H) * scale
    m = jnp.maximum(jnp.max(ln_, axis=1), ls)
    es = jnp.exp(ls - m)
    en = jnp.exp(ln_ - m[:, None, :])
    ssum = es + jnp.sum(en, axis=1)
    a_self = es / ssum
    a_n = en / ssum[:, None, :]
    af = (a_n.reshape(B * K, H) @ ST).reshape(B, K, D)
    o_ref[...] = (a_self @ ST) * vs + jnp.sum(af * vg, axis=1)


def _attn(q, ks, vs, kvg):
    grid = (NP // ABLK,)
    return pl.pallas_call(
        _attn_body,
        grid=grid,
        in_specs=[
            _row_spec(ABLK, D), _row_spec(ABLK, D), _row_spec(ABLK, D),
            _row_spec(ABLK * K, D),
        ],
        out_specs=_row_spec(ABLK, D),
        out_shape=jax.ShapeDtypeStruct((NP, D), jnp.float32),
    )(q, ks, vs, kvg)


def _post_body(x_ref, o_in, wo, bo, l2s, l2b, w1, b1, w2, b2, y_ref):
    x2 = x_ref[...] + o_in[...] @ wo[...] + bo[...]
    h2 = _lnorm(x2, l2s[...], l2b[...])
    y_ref[...] = x2 + jnp.maximum(h2 @ w1[...] + b1[...], 0.0) @ w2[...] + b2[...]


def _post(x, o, p):
    grid = (NP // BLK,)
    return pl.pallas_call(
        _post_body,
        grid=grid,
        in_specs=[
            _row_spec(BLK, D), _row_spec(BLK, D),
            _full_spec((D, D)), _full_spec((1, D)),
            _full_spec((1, D)), _full_spec((1, D)),
            _full_spec((D, FF)), _full_spec((1, FF)),
            _full_spec((FF, D)), _full_spec((1, D)),
        ],
        out_specs=_row_spec(BLK, D),
        out_shape=jax.ShapeDtypeStruct((NP, D), jnp.float32),
    )(x, o, p['Wo'], p['bo'].reshape(1, D),
      p['ln2_s'].reshape(1, D), p['ln2_b'].reshape(1, D),
      p['W1'], p['b1'].reshape(1, FF), p['W2'], p['b2'].reshape(1, D))


def _dec_body(x_ref, w1, b1, w2, b2, y_ref):
    h = jnp.maximum(x_ref[...] @ w1[...] + b1[...], 0.0)
    y_ref[...] = h @ w2[...] + b2[...]


def _dec(x, d):
    grid = (NP // BLK,)
    return pl.pallas_call(
        _dec_body,
        grid=grid,
        in_specs=[
            _row_spec(BLK, D),
            _full_spec((D, D)), _full_spec((1, D)),
            _full_spec((D, OUT)), _full_spec((1, OUT)),
        ],
        out_specs=_row_spec(BLK, OUT),
        out_shape=jax.ShapeDtypeStruct((NP, OUT), jnp.float32),
    )(x, d['W1'], d['b1'].reshape(1, D), d['W2'], d['b2'].reshape(1, OUT))


NB = 3


def _sc_gather_body(kv_hbm, idx_hbm, kvg_hbm, idx_v, kr, *sems):
    sg, so = sems[0:NB], sems[NB:2 * NB]
    wid = lax.axis_index("s") * 2 + lax.axis_index("c")
    base = wid * IPW
    pltpu.sync_copy(idx_hbm.at[pl.ds(base, IPW)], idx_v)

    pend_g = {}
    pend_o = {}
    for i in range(NCHUNK + 1):
        if i < NCHUNK:
            b = i % NB
            if i >= NB:
                pend_o.pop(i - NB).wait()
            ii = pl.ds(i * GCH, GCH)
            pend_g[i] = pltpu.async_copy(kv_hbm.at[idx_v.at[ii]], kr.at[b], sg[b])
        if i >= 1:
            j = i - 1
            b = j % NB
            pend_g.pop(j).wait()
            off = pl.ds(base + j * GCH, GCH)
            pend_o[j] = pltpu.async_copy(kr.at[b], kvg_hbm.at[off], so[b])
    for j in sorted(pend_o):
        pend_o[j].wait()


def _sc_gather(kv_all, idx_flat):
    mesh = plsc.VectorSubcoreMesh(core_axis_name="c", subcore_axis_name="s",
                                  num_cores=2, num_subcores=16)
    shp = jax.ShapeDtypeStruct((NP * K, D), jnp.int32)
    fn = pl.kernel(
        _sc_gather_body,
        out_type=shp,
        mesh=mesh,
        scratch_types=[
            pltpu.VMEM((IPW,), jnp.int32),
            pltpu.VMEM((NB, GCH, D), jnp.int32),
        ] + [pltpu.SemaphoreType.DMA] * (2 * NB),
    )
    return fn(kv_all, idx_flat)


def kernel(x, params, idx_k8):
    xp = jnp.pad(x, ((0, NP - N), (0, 0)))
    idx_flat = jnp.pad(idx_k8, ((0, NP - N), (0, 0))).reshape(NP * K)
    h = _enc(xp, params['enc'])
    for p in params['blocks']:
        q, k_all, v_all, kv_packed = _qkv(h, p)
        kvg = _sc_gather(kv_packed, idx_flat)
        o = _attn(q, k_all, v_all, kvg)
        h = _post(h, o, p)
    out = _dec(h, params['dec'])
    return out[:N]


# speedup vs baseline: 3.2918x; 1.0914x over previous
import functools

import jax
import jax.numpy as jnp
import numpy as np
from jax import lax
from jax.experimental import pallas as pl
from jax.experimental.pallas import tpu as pltpu
from jax.experimental.pallas import tpu_sc as plsc

N = 50000
D = 128
H = 4
DH = 32
K = 8
FF = 512
OUT = 128

NW = 32
NP = 50176
PW = NP // NW
IPW = PW * K
GCH = 128
NCHUNK = IPW // GCH

BLK = 512
ABLK = 256


def _lnorm(h, s, b):
    m = jnp.mean(h, axis=-1, keepdims=True)
    v = jnp.mean((h - m) ** 2, axis=-1, keepdims=True)
    return (h - m) * lax.rsqrt(v + 1e-5) * s + b


def _row_spec(blk, width):
    return pl.BlockSpec((blk, width), lambda i: (i, 0))


def _full_spec(shape):
    return pl.BlockSpec(shape, lambda i: tuple(0 for _ in shape))


def _pack_kv(k, v):
    kw = lax.bitcast_convert_type(k.astype(jnp.bfloat16), jnp.uint16).astype(jnp.uint32)
    vw = lax.bitcast_convert_type(v.astype(jnp.bfloat16), jnp.uint16).astype(jnp.uint32)
    return ((kw << 16) | vw).astype(jnp.int32)


def _unpack_kv(w):
    ww = lax.bitcast_convert_type(w, jnp.uint32)
    k = lax.bitcast_convert_type((ww >> 16).astype(jnp.uint16), jnp.bfloat16)
    v = lax.bitcast_convert_type(ww.astype(jnp.uint16), jnp.bfloat16)
    return k.astype(jnp.float32), v.astype(jnp.float32)


def _qkv_part(h, ls, lb, wq, bq, wk, bk, wv, bv):
    hn = _lnorm(h, ls[...], lb[...])
    q = hn @ wq[...] + bq[...]
    k = hn @ wk[...] + bk[...]
    v = hn @ wv[...] + bv[...]
    return q, _pack_kv(k, v)


def _enc_qkv_body(x_ref, w1, b1, w2, b2, els, elb,
                  ls, lb, wq, bq, wk, bk, wv, bv,
                  h_ref, q_ref, kv_ref):
    h = jnp.maximum(x_ref[...] @ w1[...] + b1[...], 0.0)
    h = h @ w2[...] + b2[...]
    h = _lnorm(h, els[...], elb[...])
    h_ref[...] = h
    q, kv = _qkv_part(h, ls, lb, wq, bq, wk, bk, wv, bv)
    q_ref[...] = q
    kv_ref[...] = kv


def _enc_qkv(x, e, p):
    grid = (NP // BLK,)
    shp = jax.ShapeDtypeStruct((NP, D), jnp.float32)
    shi = jax.ShapeDtypeStruct((NP, D), jnp.int32)
    return pl.pallas_call(
        _enc_qkv_body,
        grid=grid,
        in_specs=[
            _row_spec(BLK, D),
            _full_spec((D, D)), _full_spec((1, D)),
            _full_spec((D, D)), _full_spec((1, D)),
            _full_spec((1, D)), _full_spec((1, D)),
            _full_spec((1, D)), _full_spec((1, D)),
            _full_spec((D, D)), _full_spec((1, D)),
            _full_spec((D, D)), _full_spec((1, D)),
            _full_spec((D, D)), _full_spec((1, D)),
        ],
        out_specs=[_row_spec(BLK, D)] * 3,
        out_shape=[shp, shp, shi],
    )(x, e['W1'], e['b1'].reshape(1, D), e['W2'], e['b2'].reshape(1, D),
      e['ln_s'].reshape(1, D), e['ln_b'].reshape(1, D),
      p['ln1_s'].reshape(1, D), p['ln1_b'].reshape(1, D),
      p['Wq'], p['bq'].reshape(1, D), p['Wk'], p['bk'].reshape(1, D),
      p['Wv'], p['bv'].reshape(1, D))


def _attn_body(q_ref, kvs_ref, kvg_ref, o_ref):
    B = q_ref.shape[0]
    q = q_ref[...]
    ks, vs = _unpack_kv(kvs_ref[...])
    kgf, vgf = _unpack_kv(kvg_ref[...])
    kg = kgf.reshape(B, K, D)
    vg = vgf.reshape(B, K, D)
    r = lax.broadcasted_iota(jnp.int32, (D, H), 0) // DH
    c = lax.broadcasted_iota(jnp.int32, (D, H), 1)
    S = (r == c).astype(jnp.float32)
    r2 = lax.broadcasted_iota(jnp.int32, (H, D), 0)
    c2 = lax.broadcasted_iota(jnp.int32, (H, D), 1) // DH
    ST = (r2 == c2).astype(jnp.float32)
    scale = np.float32(1.0 / np.sqrt(DH))
    ls = ((q * ks) @ S) * scale
    prod = (q[:, None, :] * kg).reshape(B * K, D)
    ln_ = (prod @ S).reshape(B, K, H) * scale
    m = jnp.maximum(jnp.max(ln_, axis=1), ls)
    es = jnp.exp(ls - m)
    en = jnp.exp(ln_ - m[:, None, :])
    ssum = es + jnp.sum(en, axis=1)
    a_self = es / ssum
    a_n = en / ssum[:, None, :]
    af = (a_n.reshape(B * K, H) @ ST).reshape(B, K, D)
    o_ref[...] = (a_self @ ST) * vs + jnp.sum(af * vg, axis=1)


def _attn(q, kvs, kvg):
    grid = (NP // ABLK,)
    return pl.pallas_call(
        _attn_body,
        grid=grid,
        in_specs=[
            _row_spec(ABLK, D), _row_spec(ABLK, D),
            _row_spec(ABLK * K, D),
        ],
        out_specs=_row_spec(ABLK, D),
        out_shape=jax.ShapeDtypeStruct((NP, D), jnp.float32),
    )(q, kvs, kvg)


def _ff_part(x, o, wo, bo, l2s, l2b, w1, b1, w2, b2):
    x2 = x + o @ wo[...] + bo[...]
    h2 = _lnorm(x2, l2s[...], l2b[...])
    return x2 + jnp.maximum(h2 @ w1[...] + b1[...], 0.0) @ w2[...] + b2[...]


def _post_qkv_body(x_ref, o_in, wo, bo, l2s, l2b, w1, b1, w2, b2,
                   ls, lb, wq, bq, wk, bk, wv, bv,
                   x2_ref, q_ref, kv_ref):
    y = _ff_part(x_ref[...], o_in[...], wo, bo, l2s, l2b, w1, b1, w2, b2)
    x2_ref[...] = y
    q, kv = _qkv_part(y, ls, lb, wq, bq, wk, bk, wv, bv)
    q_ref[...] = q
    kv_ref[...] = kv


def _post_qkv(x, o, p, p2):
    grid = (NP // BLK,)
    shp = jax.ShapeDtypeStruct((NP, D), jnp.float32)
    shi = jax.ShapeDtypeStruct((NP, D), jnp.int32)
    return pl.pallas_call(
        _post_qkv_body,
        grid=grid,
        in_specs=[
            _row_spec(BLK, D), _row_spec(BLK, D),
            _full_spec((D, D)), _full_spec((1, D)),
            _full_spec((1, D)), _full_spec((1, D)),
            _full_spec((D, FF)), _full_spec((1, FF)),
            _full_spec((FF, D)), _full_spec((1, D)),
            _full_spec((1, D)), _full_spec((1, D)),
            _full_spec((D, D)), _full_spec((1, D)),
            _full_spec((D, D)), _full_spec((1, D)),
            _full_spec((D, D)), _full_spec((1, D)),
        ],
        out_specs=[_row_spec(BLK, D)] * 3,
        out_shape=[shp, shp, shi],
    )(x, o, p['Wo'], p['bo'].reshape(1, D),
      p['ln2_s'].reshape(1, D), p['ln2_b'].reshape(1, D),
      p['W1'], p['b1'].reshape(1, FF), p['W2'], p['b2'].reshape(1, D),
      p2['ln1_s'].reshape(1, D), p2['ln1_b'].reshape(1, D),
      p2['Wq'], p2['bq'].reshape(1, D), p2['Wk'], p2['bk'].reshape(1, D),
      p2['Wv'], p2['bv'].reshape(1, D))


def _post_dec_body(x_ref, o_in, wo, bo, l2s, l2b, w1, b1, w2, b2,
                   dw1, db1, dw2, db2, y_ref):
    y = _ff_part(x_ref[...], o_in[...], wo, bo, l2s, l2b, w1, b1, w2, b2)
    h = jnp.maximum(y @ dw1[...] + db1[...], 0.0)
    y_ref[...] = h @ dw2[...] + db2[...]


def _post_dec(x, o, p, d):
    grid = (NP // BLK,)
    return pl.pallas_call(
        _post_dec_body,
        grid=grid,
        in_specs=[
            _row_spec(BLK, D), _row_spec(BLK, D),
            _full_spec((D, D)), _full_spec((1, D)),
            _full_spec((1, D)), _full_spec((1, D)),
            _full_spec((D, FF)), _full_spec((1, FF)),
            _full_spec((FF, D)), _full_spec((1, D)),
            _full_spec((D, D)), _full_spec((1, D)),
            _full_spec((D, OUT)), _full_spec((1, OUT)),
        ],
        out_specs=_row_spec(BLK, OUT),
        out_shape=jax.ShapeDtypeStruct((NP, OUT), jnp.float32),
    )(x, o, p['Wo'], p['bo'].reshape(1, D),
      p['ln2_s'].reshape(1, D), p['ln2_b'].reshape(1, D),
      p['W1'], p['b1'].reshape(1, FF), p['W2'], p['b2'].reshape(1, D),
      d['W1'], d['b1'].reshape(1, D), d['W2'], d['b2'].reshape(1, OUT))


NB = 3


def _sc_gather_body(kv_hbm, idx_hbm, kvg_hbm, idx_v, kr, *sems):
    sg, so = sems[0:NB], sems[NB:2 * NB]
    wid = lax.axis_index("s") * 2 + lax.axis_index("c")
    base = wid * IPW
    pltpu.sync_copy(idx_hbm.at[pl.ds(base, IPW)], idx_v)

    pend_g = {}
    pend_o = {}
    for i in range(NCHUNK + 1):
        if i < NCHUNK:
            b = i % NB
            if i >= NB:
                pend_o.pop(i - NB).wait()
            ii = pl.ds(i * GCH, GCH)
            pend_g[i] = pltpu.async_copy(kv_hbm.at[idx_v.at[ii]], kr.at[b], sg[b])
        if i >= 1:
            j = i - 1
            b = j % NB
            pend_g.pop(j).wait()
            off = pl.ds(base + j * GCH, GCH)
            pend_o[j] = pltpu.async_copy(kr.at[b], kvg_hbm.at[off], so[b])
    for j in sorted(pend_o):
        pend_o[j].wait()


def _sc_gather(kv_all, idx_flat):
    mesh = plsc.VectorSubcoreMesh(core_axis_name="c", subcore_axis_name="s",
                                  num_cores=2, num_subcores=16)
    shp = jax.ShapeDtypeStruct((NP * K, D), jnp.int32)
    fn = pl.kernel(
        _sc_gather_body,
        out_type=shp,
        mesh=mesh,
        scratch_types=[
            pltpu.VMEM((IPW,), jnp.int32),
            pltpu.VMEM((NB, GCH, D), jnp.int32),
        ] + [pltpu.SemaphoreType.DMA] * (2 * NB),
    )
    return fn(kv_all, idx_flat)


def kernel(x, params, idx_k8):
    xp = jnp.pad(x, ((0, NP - N), (0, 0)))
    idx_flat = jnp.pad(idx_k8, ((0, NP - N), (0, 0))).reshape(NP * K)
    p0, p1 = params['blocks']
    h, q, kv = _enc_qkv(xp, params['enc'], p0)
    kvg = _sc_gather(kv, idx_flat)
    o = _attn(q, kv, kvg)
    x2, q2, kv2 = _post_qkv(h, o, p0, p1)
    kvg2 = _sc_gather(kv2, idx_flat)
    o2 = _attn(q2, kv2, kvg2)
    y = _post_dec(x2, o2, p1, params['dec'])
    return y[:N]


# speedup vs baseline: 4.0672x; 1.2356x over previous
import functools

import jax
import jax.numpy as jnp
import numpy as np
from jax import lax
from jax.experimental import pallas as pl
from jax.experimental.pallas import tpu as pltpu
from jax.experimental.pallas import tpu_sc as plsc

N = 50000
D = 128
H = 4
DH = 32
K = 8
FF = 512
OUT = 128

NW = 32
NP = 50176
PW = NP // NW
IPW = PW * K
GCH = 128
NCHUNK = IPW // GCH

BLK = 512
ABLK = 256


def _lnorm(h, s, b):
    m = jnp.mean(h, axis=-1, keepdims=True)
    v = jnp.mean((h - m) ** 2, axis=-1, keepdims=True)
    return (h - m) * lax.rsqrt(v + 1e-5) * s + b


def _row_spec(blk, width):
    return pl.BlockSpec((blk, width), lambda i: (i, 0))


def _full_spec(shape):
    return pl.BlockSpec(shape, lambda i: tuple(0 for _ in shape))


def _pack_kv(k, v):
    kw = lax.bitcast_convert_type(k.astype(jnp.bfloat16), jnp.uint16).astype(jnp.uint32)
    vw = lax.bitcast_convert_type(v.astype(jnp.bfloat16), jnp.uint16).astype(jnp.uint32)
    return ((kw << 16) | vw).astype(jnp.int32)


def _unpack_kv(w):
    ww = lax.bitcast_convert_type(w, jnp.uint32)
    k = lax.bitcast_convert_type(ww & jnp.uint32(0xFFFF0000), jnp.float32)
    v = lax.bitcast_convert_type(ww << 16, jnp.float32)
    return k, v


def _qkv_part(h, ls, lb, wq, bq, wk, bk, wv, bv):
    hn = _lnorm(h, ls[...], lb[...])
    q = hn @ wq[...] + bq[...]
    k = hn @ wk[...] + bk[...]
    v = hn @ wv[...] + bv[...]
    return q, _pack_kv(k, v)


def _enc_qkv_body(x_ref, w1, b1, w2, b2, els, elb,
                  ls, lb, wq, bq, wk, bk, wv, bv,
                  h_ref, q_ref, kv_ref):
    h = jnp.maximum(x_ref[...] @ w1[...] + b1[...], 0.0)
    h = h @ w2[...] + b2[...]
    h = _lnorm(h, els[...], elb[...])
    h_ref[...] = h
    q, kv = _qkv_part(h, ls, lb, wq, bq, wk, bk, wv, bv)
    q_ref[...] = q
    kv_ref[...] = kv


def _enc_qkv(x, e, p):
    grid = (NP // BLK,)
    shp = jax.ShapeDtypeStruct((NP, D), jnp.float32)
    shi = jax.ShapeDtypeStruct((NP, D), jnp.int32)
    return pl.pallas_call(
        _enc_qkv_body,
        grid=grid,
        in_specs=[
            _row_spec(BLK, D),
            _full_spec((D, D)), _full_spec((1, D)),
            _full_spec((D, D)), _full_spec((1, D)),
            _full_spec((1, D)), _full_spec((1, D)),
            _full_spec((1, D)), _full_spec((1, D)),
            _full_spec((D, D)), _full_spec((1, D)),
            _full_spec((D, D)), _full_spec((1, D)),
            _full_spec((D, D)), _full_spec((1, D)),
        ],
        out_specs=[_row_spec(BLK, D)] * 3,
        out_shape=[shp, shp, shi],
    )(x, e['W1'], e['b1'].reshape(1, D), e['W2'], e['b2'].reshape(1, D),
      e['ln_s'].reshape(1, D), e['ln_b'].reshape(1, D),
      p['ln1_s'].reshape(1, D), p['ln1_b'].reshape(1, D),
      p['Wq'], p['bq'].reshape(1, D), p['Wk'], p['bk'].reshape(1, D),
      p['Wv'], p['bv'].reshape(1, D))


def _attn_body(q_ref, kvs_ref, kvg_ref, o_ref):
    q = q_ref[...]
    r = lax.broadcasted_iota(jnp.int32, (D, H), 0) // DH
    c = lax.broadcasted_iota(jnp.int32, (D, H), 1)
    S = (r == c).astype(jnp.float32)
    r2 = lax.broadcasted_iota(jnp.int32, (H, D), 0)
    c2 = lax.broadcasted_iota(jnp.int32, (H, D), 1) // DH
    ST = (r2 == c2).astype(jnp.float32)
    scale = np.float32(1.0 / np.sqrt(DH))
    logit = []
    vals = []
    for j in range(K + 1):
        kj, vj = _unpack_kv(kvs_ref[...] if j == 0 else kvg_ref[j - 1])
        logit.append(((q * kj) @ S) * scale)
        vals.append(vj)
    m = logit[0]
    for x in logit[1:]:
        m = jnp.maximum(m, x)
    es = [jnp.exp(x - m) for x in logit]
    ssum = es[0]
    for e in es[1:]:
        ssum = ssum + e
    rinv = 1.0 / ssum
    o = ((es[0] * rinv) @ ST) * vals[0]
    for j in range(1, K + 1):
        o = o + ((es[j] * rinv) @ ST) * vals[j]
    o_ref[...] = o


def _attn(q, kvs, kvg):
    grid = (NP // ABLK,)
    return pl.pallas_call(
        _attn_body,
        grid=grid,
        in_specs=[
            _row_spec(ABLK, D), _row_spec(ABLK, D),
            pl.BlockSpec((K, ABLK, D), lambda i: (0, i, 0)),
        ],
        out_specs=_row_spec(ABLK, D),
        out_shape=jax.ShapeDtypeStruct((NP, D), jnp.float32),
    )(q, kvs, kvg)


def _ff_part(x, o, wo, bo, l2s, l2b, w1, b1, w2, b2):
    x2 = x + o @ wo[...] + bo[...]
    h2 = _lnorm(x2, l2s[...], l2b[...])
    return x2 + jnp.maximum(h2 @ w1[...] + b1[...], 0.0) @ w2[...] + b2[...]


def _post_qkv_body(x_ref, o_in, wo, bo, l2s, l2b, w1, b1, w2, b2,
                   ls, lb, wq, bq, wk, bk, wv, bv,
                   x2_ref, q_ref, kv_ref):
    y = _ff_part(x_ref[...], o_in[...], wo, bo, l2s, l2b, w1, b1, w2, b2)
    x2_ref[...] = y
    q, kv = _qkv_part(y, ls, lb, wq, bq, wk, bk, wv, bv)
    q_ref[...] = q
    kv_ref[...] = kv


def _post_qkv(x, o, p, p2):
    grid = (NP // BLK,)
    shp = jax.ShapeDtypeStruct((NP, D), jnp.float32)
    shi = jax.ShapeDtypeStruct((NP, D), jnp.int32)
    return pl.pallas_call(
        _post_qkv_body,
        grid=grid,
        in_specs=[
            _row_spec(BLK, D), _row_spec(BLK, D),
            _full_spec((D, D)), _full_spec((1, D)),
            _full_spec((1, D)), _full_spec((1, D)),
            _full_spec((D, FF)), _full_spec((1, FF)),
            _full_spec((FF, D)), _full_spec((1, D)),
            _full_spec((1, D)), _full_spec((1, D)),
            _full_spec((D, D)), _full_spec((1, D)),
            _full_spec((D, D)), _full_spec((1, D)),
            _full_spec((D, D)), _full_spec((1, D)),
        ],
        out_specs=[_row_spec(BLK, D)] * 3,
        out_shape=[shp, shp, shi],
    )(x, o, p['Wo'], p['bo'].reshape(1, D),
      p['ln2_s'].reshape(1, D), p['ln2_b'].reshape(1, D),
      p['W1'], p['b1'].reshape(1, FF), p['W2'], p['b2'].reshape(1, D),
      p2['ln1_s'].reshape(1, D), p2['ln1_b'].reshape(1, D),
      p2['Wq'], p2['bq'].reshape(1, D), p2['Wk'], p2['bk'].reshape(1, D),
      p2['Wv'], p2['bv'].reshape(1, D))


def _post_dec_body(x_ref, o_in, wo, bo, l2s, l2b, w1, b1, w2, b2,
                   dw1, db1, dw2, db2, y_ref):
    y = _ff_part(x_ref[...], o_in[...], wo, bo, l2s, l2b, w1, b1, w2, b2)
    h = jnp.maximum(y @ dw1[...] + db1[...], 0.0)
    y_ref[...] = h @ dw2[...] + db2[...]


def _post_dec(x, o, p, d):
    grid = (NP // BLK,)
    return pl.pallas_call(
        _post_dec_body,
        grid=grid,
        in_specs=[
            _row_spec(BLK, D), _row_spec(BLK, D),
            _full_spec((D, D)), _full_spec((1, D)),
            _full_spec((1, D)), _full_spec((1, D)),
            _full_spec((D, FF)), _full_spec((1, FF)),
            _full_spec((FF, D)), _full_spec((1, D)),
            _full_spec((D, D)), _full_spec((1, D)),
            _full_spec((D, OUT)), _full_spec((1, OUT)),
        ],
        out_specs=_row_spec(BLK, OUT),
        out_shape=jax.ShapeDtypeStruct((NP, OUT), jnp.float32),
    )(x, o, p['Wo'], p['bo'].reshape(1, D),
      p['ln2_s'].reshape(1, D), p['ln2_b'].reshape(1, D),
      p['W1'], p['b1'].reshape(1, FF), p['W2'], p['b2'].reshape(1, D),
      d['W1'], d['b1'].reshape(1, D), d['W2'], d['b2'].reshape(1, OUT))


NB = 3


def _sc_gather_body(kv_hbm, idx_hbm, kvg_hbm, idx_v, kr, *sems):
    sg, so = sems[0:NB], sems[NB:2 * NB]
    wid = lax.axis_index("s") * 2 + lax.axis_index("c")
    base = wid * IPW
    pltpu.sync_copy(idx_hbm.at[pl.ds(base, IPW)], idx_v)

    pend_g = {}
    pend_o = {}
    for i in range(NCHUNK + 1):
        if i < NCHUNK:
            b = i % NB
            if i >= NB:
                pend_o.pop(i - NB).wait()
            ii = pl.ds(i * GCH, GCH)
            pend_g[i] = pltpu.async_copy(kv_hbm.at[idx_v.at[ii]], kr.at[b], sg[b])
        if i >= 1:
            j = i - 1
            b = j % NB
            pend_g.pop(j).wait()
            off = pl.ds(base + j * GCH, GCH)
            pend_o[j] = pltpu.async_copy(kr.at[b], kvg_hbm.at[off], so[b])
    for j in sorted(pend_o):
        pend_o[j].wait()


def _sc_gather(kv_all, idx_flat):
    mesh = plsc.VectorSubcoreMesh(core_axis_name="c", subcore_axis_name="s",
                                  num_cores=2, num_subcores=16)
    shp = jax.ShapeDtypeStruct((NP * K, D), jnp.int32)
    fn = pl.kernel(
        _sc_gather_body,
        out_type=shp,
        mesh=mesh,
        scratch_types=[
            pltpu.VMEM((IPW,), jnp.int32),
            pltpu.VMEM((NB, GCH, D), jnp.int32),
        ] + [pltpu.SemaphoreType.DMA] * (2 * NB),
    )
    return fn(kv_all, idx_flat)


def kernel(x, params, idx_k8):
    xp = jnp.pad(x, ((0, NP - N), (0, 0)))
    idx_flat = jnp.pad(idx_k8, ((0, NP - N), (0, 0))).T.reshape(NP * K)
    p0, p1 = params['blocks']
    h, q, kv = _enc_qkv(xp, params['enc'], p0)
    kvg = _sc_gather(kv, idx_flat).reshape(K, NP, D)
    o = _attn(q, kv, kvg)
    x2, q2, kv2 = _post_qkv(h, o, p0, p1)
    kvg2 = _sc_gather(kv2, idx_flat).reshape(K, NP, D)
    o2 = _attn(q2, kv2, kvg2)
    y = _post_dec(x2, o2, p1, params['dec'])
    return y[:N]
